# baseline (speedup 1.0000x reference)
import functools

import jax
import jax.numpy as jnp
from jax import lax
from jax.experimental import pallas as pl
from jax.experimental.pallas import tpu as pltpu
from jax.experimental.pallas import tpu_sc as plsc

N_NODES = 10000
D_IN = 128
D_HID = 128
D_EMB = 64
N_EDGES = 320000

NP = 10240
NC = 2
NS = 16
NW = NC * NS
K = 128
CH = 80
EP = NW * CH * K
RPT = NP // NS
NPARTS = RPT // K

_mesh = plsc.VectorSubcoreMesh(core_axis_name="c", subcore_axis_name="s")


def _fill(buf, rows, value):
  @pl.loop(0, rows)
  def _(r):
    vec = jnp.full((16,), value, jnp.float32)
    for cc in range(8):
      buf[r, pl.ds(cc * 16, 16)] = vec


def _deg_body(dst_hbm, out_hbm, dstv, buf, shared):
  c = lax.axis_index("c")
  s = lax.axis_index("s")
  wid = c * NS + s

  _fill(buf, K, 0.0)
  for part in range(NPARTS):
    pltpu.sync_copy(buf, shared.at[pl.ds(s * RPT + part * K, K)])
  _fill(buf, K, 1.0)
  plsc.subcore_barrier()

  pltpu.sync_copy(dst_hbm.at[wid], dstv)

  @pl.loop(0, CH)
  def _(j):
    pltpu.sync_copy(buf, shared.at[dstv.at[j]], add=True)

  plsc.subcore_barrier()
  for part in range(NPARTS):
    pltpu.sync_copy(shared.at[pl.ds(s * RPT + part * K, K)], buf)
    pltpu.sync_copy(buf, out_hbm.at[pl.ds(c * NP + s * RPT + part * K, K)])


_deg_kernel = functools.partial(
    pl.kernel,
    out_type=jax.ShapeDtypeStruct((NC * NP, 128), jnp.float32),
    mesh=_mesh,
    scratch_types=[
        pltpu.VMEM((CH, K), jnp.int32),
        pltpu.VMEM((K, 128), jnp.float32),
        pltpu.VMEM_SHARED((NP, 128), jnp.float32),
    ],
)(_deg_body)


KE = 64
NPH = 2
PCH = 80
CHT = NPH * PCH
NPARTS2 = RPT // KE
NBUF = 4


def _spmm_body(xp_hbm, src_hbm, dst_hbm, out_hbm, srcv, dstv,
               b0, b1, b2, b3, s0, s1, s2, s3, x0, x1, x2, x3, shared):
  bufs = (b0, b1, b2, b3)
  sems = (s0, s1, s2, s3)
  ssems = (x0, x1, x2, x3)
  c = lax.axis_index("c")
  s = lax.axis_index("s")
  wid = c * NS + s

  _fill(b0, KE, 0.0)
  for part in range(NPARTS2):
    pltpu.sync_copy(b0, shared.at[pl.ds(s * RPT + part * KE, KE)])
  plsc.subcore_barrier()

  for h in range(NPH):
    pltpu.sync_copy(src_hbm.at[wid, pl.ds(h * (PCH // 2), PCH // 2)], srcv)
    pltpu.sync_copy(dst_hbm.at[wid, pl.ds(h * PCH, PCH)], dstv)
    for t in range(NBUF - 1):
      pltpu.async_copy(
          xp_hbm.at[srcv.at[t // 2, pl.ds((t % 2) * KE, KE)]],
          bufs[t], sems[t])

    @pl.loop(0, PCH // 2, step=2)
    def _(q):
      for t in range(NBUF):
        pltpu.make_async_copy(
            xp_hbm.at[srcv.at[q + t // 2, pl.ds((t % 2) * KE, KE)]],
            bufs[t], sems[t]).wait()

        @pl.when(2 * q + t + NBUF - 1 < PCH)
        def _():
          tn = (t + NBUF - 1) % NBUF
          @pl.when(2 * q + t > 0)
          def _():
            pltpu.make_async_copy(
                bufs[tn], shared.at[dstv.at[2 * q + t - 1]], ssems[tn]).wait()
          pltpu.async_copy(
              xp_hbm.at[srcv.at[q + (t + NBUF - 1) // 2,
                                pl.ds(((t + NBUF - 1) % 2) * KE, KE)]],
              bufs[tn], sems[tn])

        pltpu.async_copy(
            bufs[t], shared.at[dstv.at[2 * q + t]], ssems[t], add=True)

    for t in range(NBUF):
      pltpu.make_async_copy(
          bufs[t], shared.at[dstv.at[0]], ssems[t]).wait()

  plsc.subcore_barrier()
  for part in range(NPARTS2):
    pltpu.sync_copy(shared.at[pl.ds(s * RPT + part * KE, KE)], b0)
    pltpu.sync_copy(b0, out_hbm.at[pl.ds(c * NP + s * RPT + part * KE, KE)])


_spmm128 = functools.partial(
    pl.kernel,
    out_type=jax.ShapeDtypeStruct((NC * NP, D_HID), jnp.float32),
    mesh=_mesh,
    scratch_types=[
        pltpu.VMEM((CHT // 4, 2 * KE), jnp.int32),
        pltpu.VMEM((PCH, KE), jnp.int32),
        pltpu.VMEM((KE, D_HID), jnp.float32),
        pltpu.VMEM((KE, D_HID), jnp.float32),
        pltpu.VMEM((KE, D_HID), jnp.float32),
        pltpu.VMEM((KE, D_HID), jnp.float32),
        pltpu.SemaphoreType.DMA,
        pltpu.SemaphoreType.DMA,
        pltpu.SemaphoreType.DMA,
        pltpu.SemaphoreType.DMA,
        pltpu.SemaphoreType.DMA,
        pltpu.SemaphoreType.DMA,
        pltpu.SemaphoreType.DMA,
        pltpu.SemaphoreType.DMA,
        pltpu.VMEM_SHARED((NP, D_HID), jnp.float32),
    ],
)(_spmm_body)


_mesh1 = plsc.VectorSubcoreMesh(core_axis_name="c", subcore_axis_name="s",
                                num_cores=1)
NPH1 = 4
CHT1 = NPH1 * PCH


def _spmm1_body(xp_hbm, src_hbm, dst_hbm, out_hbm, srcv, dstv,
                b0, b1, b2, b3, s0, s1, s2, s3, x0, x1, x2, x3, shared):
  bufs = (b0, b1, b2, b3)
  sems = (s0, s1, s2, s3)
  ssems = (x0, x1, x2, x3)
  s = lax.axis_index("s")
  wid = s

  _fill(b0, KE, 0.0)
  for part in range(NPARTS2):
    pltpu.sync_copy(b0, shared.at[pl.ds(s * RPT + part * KE, KE)])
  plsc.subcore_barrier()

  for h in range(NPH1):
    pltpu.sync_copy(src_hbm.at[wid, pl.ds(h * (PCH // 2), PCH // 2)], srcv)
    pltpu.sync_copy(dst_hbm.at[wid, pl.ds(h * PCH, PCH)], dstv)
    for t in range(NBUF - 1):
      pltpu.async_copy(
          xp_hbm.at[srcv.at[t // 2, pl.ds((t % 2) * KE, KE)]],
          bufs[t], sems[t])

    @pl.loop(0, PCH // 2, step=2)
    def _(q):
      for t in range(NBUF):
        pltpu.make_async_copy(
            xp_hbm.at[srcv.at[q + t // 2, pl.ds((t % 2) * KE, KE)]],
            bufs[t], sems[t]).wait()

        @pl.when(2 * q + t + NBUF - 1 < PCH)
        def _():
          tn = (t + NBUF - 1) % NBUF
          @pl.when(2 * q + t > 0)
          def _():
            pltpu.make_async_copy(
                bufs[tn], shared.at[dstv.at[2 * q + t - 1]], ssems[tn]).wait()
          pltpu.async_copy(
              xp_hbm.at[srcv.at[q + (t + NBUF - 1) // 2,
                                pl.ds(((t + NBUF - 1) % 2) * KE, KE)]],
              bufs[tn], sems[tn])

        pltpu.async_copy(
            bufs[t], shared.at[dstv.at[2 * q + t]], ssems[t], add=True)

    for t in range(NBUF):
      pltpu.make_async_copy(
          bufs[t], shared.at[dstv.at[0]], ssems[t]).wait()

  plsc.subcore_barrier()
  for part in range(NPARTS2):
    pltpu.sync_copy(shared.at[pl.ds(s * RPT + part * KE, KE)], b0)
    pltpu.sync_copy(b0, out_hbm.at[pl.ds(s * RPT + part * KE, KE)])


_spmm1 = functools.partial(
    pl.kernel,
    out_type=jax.ShapeDtypeStruct((NP, D_HID), jnp.float32),
    mesh=_mesh1,
    scratch_types=[
        pltpu.VMEM((PCH // 2, 2 * KE), jnp.int32),
        pltpu.VMEM((PCH, KE), jnp.int32),
        pltpu.VMEM((KE, D_HID), jnp.float32),
        pltpu.VMEM((KE, D_HID), jnp.float32),
        pltpu.VMEM((KE, D_HID), jnp.float32),
        pltpu.VMEM((KE, D_HID), jnp.float32),
        pltpu.SemaphoreType.DMA,
        pltpu.SemaphoreType.DMA,
        pltpu.SemaphoreType.DMA,
        pltpu.SemaphoreType.DMA,
        pltpu.SemaphoreType.DMA,
        pltpu.SemaphoreType.DMA,
        pltpu.SemaphoreType.DMA,
        pltpu.SemaphoreType.DMA,
        pltpu.VMEM_SHARED((NP, D_HID), jnp.float32),
    ],
)(_spmm1_body)


_BR = 512


def _dis_body(d0_ref, d1_ref, mask_ref, out_ref):
  deg = d0_ref[...] + d1_ref[...] + 1.0
  out_ref[...] = mask_ref[...] * lax.rsqrt(deg)


def _dis_kernel(d0, d1, mask):
  return pl.pallas_call(
      _dis_body,
      out_shape=jax.ShapeDtypeStruct((NP, 1), jnp.float32),
  )(d0, d1, mask)


def _pre_body(a_ref, w_ref, dis_ref, out_ref):
  h = jnp.dot(a_ref[...], w_ref[...], preferred_element_type=jnp.float32)
  out_ref[...] = h * dis_ref[...]


def _pre_kernel(a, w, dis):
  din, dout = w.shape
  return pl.pallas_call(
      _pre_body,
      grid=(NP // _BR,),
      in_specs=[
          pl.BlockSpec((_BR, din), lambda i: (i, 0)),
          pl.BlockSpec((din, dout), lambda i: (0, 0)),
          pl.BlockSpec((_BR, 1), lambda i: (i, 0)),
      ],
      out_specs=pl.BlockSpec((_BR, dout), lambda i: (i, 0)),
      out_shape=jax.ShapeDtypeStruct((NP, dout), jnp.float32),
  )(a, w, dis)


def _mid_body(z0_ref, z1_ref, xp_ref, dis_ref, b_ref, w_ref, out_ref):
  h = dis_ref[...] * (z0_ref[...] + z1_ref[...] + xp_ref[...]) + b_ref[...]
  a = jnp.maximum(h, 0.0)
  out_ref[...] = (
      jnp.dot(a, w_ref[...], preferred_element_type=jnp.float32)
      * dis_ref[...])


def _mid_kernel(z0, z1, xp, dis, b, w):
  din, dout = w.shape
  return pl.pallas_call(
      _mid_body,
      grid=(NP // _BR,),
      in_specs=[
          pl.BlockSpec((_BR, din), lambda i: (i, 0)),
          pl.BlockSpec((_BR, din), lambda i: (i, 0)),
          pl.BlockSpec((_BR, din), lambda i: (i, 0)),
          pl.BlockSpec((_BR, 1), lambda i: (i, 0)),
          pl.BlockSpec((1, din), lambda i: (0, 0)),
          pl.BlockSpec((din, dout), lambda i: (0, 0)),
      ],
      out_specs=pl.BlockSpec((_BR, dout), lambda i: (i, 0)),
      out_shape=jax.ShapeDtypeStruct((NP, dout), jnp.float32),
  )(z0, z1, xp, dis, b, w)


def _final_body(z0_ref, z1_ref, xp_ref, dis_ref, b_ref, out_ref):
  h = dis_ref[...] * (z0_ref[...] + z1_ref[...] + xp_ref[...]) + b_ref[...]
  nrm = jnp.sqrt(jnp.sum(h * h, axis=1, keepdims=True))
  out_ref[...] = h / jnp.maximum(nrm, 1e-12)


def _final_kernel(z0, z1, xp, dis, b):
  d = b.shape[1]
  return pl.pallas_call(
      _final_body,
      grid=(NP // _BR,),
      in_specs=[
          pl.BlockSpec((_BR, d), lambda i: (i, 0)),
          pl.BlockSpec((_BR, d), lambda i: (i, 0)),
          pl.BlockSpec((_BR, d), lambda i: (i, 0)),
          pl.BlockSpec((_BR, 1), lambda i: (i, 0)),
          pl.BlockSpec((1, d), lambda i: (0, 0)),
      ],
      out_specs=pl.BlockSpec((_BR, d), lambda i: (i, 0)),
      out_shape=jax.ShapeDtypeStruct((NP, d), jnp.float32),
  )(z0, z1, xp, dis, b)


def kernel(x, edge_index, W1, b1, W2, b2, W3, b3):
  ei = edge_index.astype(jnp.int32)
  pad = jnp.full((EP - N_EDGES,), N_NODES, jnp.int32)
  dstp_deg = jnp.concatenate([ei[1], pad]).reshape(NW, CH, K)
  srcp = jnp.concatenate([ei[0], pad]).reshape(NS, CHT1 // 2, 2 * KE)
  dstp = jnp.concatenate([ei[1], pad]).reshape(NS, CHT1, KE)
  zeros_z = jnp.zeros((NP, D_HID), jnp.float32)

  xpad = jnp.pad(x, ((0, NP - N_NODES), (0, 0)))
  mask = (jnp.arange(NP) < N_NODES).astype(jnp.float32).reshape(NP, 1)

  degp = _deg_kernel(dstp_deg)
  dis = _dis_kernel(degp[:NP, :1], degp[NP:, :1], mask)

  xp1 = _pre_kernel(xpad, W1, dis)
  zz = _spmm1(xp1, srcp, dstp)
  xp2 = _mid_kernel(zz, zeros_z, xp1, dis, b1.reshape(1, -1), W2)
  zz = _spmm1(xp2, srcp, dstp)
  W3p = jnp.pad(W3, ((0, 0), (0, D_HID - D_EMB)))
  xp3 = _mid_kernel(zz, zeros_z, xp2, dis, b2.reshape(1, -1), W3p)
  zz = _spmm1(xp3, srcp, dstp)
  emb = _final_kernel(zz[:, :D_EMB], zeros_z[:, :D_EMB], xp3[:, :D_EMB],
                      dis, b3.reshape(1, -1))
  return emb[:N_NODES]


# speedup vs baseline: 1.1232x; 1.1232x over previous
import functools

import jax
import jax.numpy as jnp
from jax import lax
from jax.experimental import pallas as pl
from jax.experimental.pallas import tpu as pltpu
from jax.experimental.pallas import tpu_sc as plsc

N_NODES = 10000
D_IN = 128
D_HID = 128
D_EMB = 64
N_EDGES = 320000

NP = 10240
NC = 2
NS = 16
NW = NC * NS
K = 128
CH = 80
EP = NW * CH * K
RPT = NP // NS
NPARTS = RPT // K

_mesh = plsc.VectorSubcoreMesh(core_axis_name="c", subcore_axis_name="s")


def _fill(buf, rows, value):
  @pl.loop(0, rows)
  def _(r):
    vec = jnp.full((16,), value, jnp.float32)
    for cc in range(8):
      buf[r, pl.ds(cc * 16, 16)] = vec


def _deg_body(dst_hbm, out_hbm, dstv, buf, shared):
  c = lax.axis_index("c")
  s = lax.axis_index("s")
  wid = c * NS + s

  _fill(buf, K, 0.0)
  for part in range(NPARTS):
    pltpu.sync_copy(buf, shared.at[pl.ds(s * RPT + part * K, K)])
  _fill(buf, K, 1.0)
  plsc.subcore_barrier()

  pltpu.sync_copy(dst_hbm.at[wid], dstv)

  @pl.loop(0, CH)
  def _(j):
    pltpu.sync_copy(buf, shared.at[dstv.at[j]], add=True)

  plsc.subcore_barrier()
  for part in range(NPARTS):
    pltpu.sync_copy(shared.at[pl.ds(s * RPT + part * K, K)], buf)
    pltpu.sync_copy(buf, out_hbm.at[pl.ds(c * NP + s * RPT + part * K, K)])


_deg_kernel = functools.partial(
    pl.kernel,
    out_type=jax.ShapeDtypeStruct((NC * NP, 128), jnp.float32),
    mesh=_mesh,
    scratch_types=[
        pltpu.VMEM((CH, K), jnp.int32),
        pltpu.VMEM((K, 128), jnp.float32),
        pltpu.VMEM_SHARED((NP, 128), jnp.float32),
    ],
)(_deg_body)


KE = 64
NPH = 2
PCH = 80
CHT = NPH * PCH
NPARTS2 = RPT // KE
NBUF = 4


def _spmm_body(xp_hbm, src_hbm, dst_hbm, out_hbm, srcv, dstv,
               b0, b1, b2, b3, s0, s1, s2, s3, x0, x1, x2, x3, shared):
  bufs = (b0, b1, b2, b3)
  sems = (s0, s1, s2, s3)
  ssems = (x0, x1, x2, x3)
  c = lax.axis_index("c")
  s = lax.axis_index("s")
  wid = c * NS + s

  _fill(b0, KE, 0.0)
  for part in range(NPARTS2):
    pltpu.sync_copy(b0, shared.at[pl.ds(s * RPT + part * KE, KE)])
  plsc.subcore_barrier()

  for h in range(NPH):
    pltpu.sync_copy(src_hbm.at[wid, pl.ds(h * (PCH // 2), PCH // 2)], srcv)
    pltpu.sync_copy(dst_hbm.at[wid, pl.ds(h * PCH, PCH)], dstv)
    for t in range(NBUF - 1):
      pltpu.async_copy(
          xp_hbm.at[srcv.at[t // 2, pl.ds((t % 2) * KE, KE)]],
          bufs[t], sems[t])

    @pl.loop(0, PCH // 2, step=2)
    def _(q):
      for t in range(NBUF):
        pltpu.make_async_copy(
            xp_hbm.at[srcv.at[q + t // 2, pl.ds((t % 2) * KE, KE)]],
            bufs[t], sems[t]).wait()

        @pl.when(2 * q + t + NBUF - 1 < PCH)
        def _():
          tn = (t + NBUF - 1) % NBUF
          @pl.when(2 * q + t > 0)
          def _():
            pltpu.make_async_copy(
                bufs[tn], shared.at[dstv.at[2 * q + t - 1]], ssems[tn]).wait()
          pltpu.async_copy(
              xp_hbm.at[srcv.at[q + (t + NBUF - 1) // 2,
                                pl.ds(((t + NBUF - 1) % 2) * KE, KE)]],
              bufs[tn], sems[tn])

        pltpu.async_copy(
            bufs[t], shared.at[dstv.at[2 * q + t]], ssems[t], add=True)

    for t in range(NBUF):
      pltpu.make_async_copy(
          bufs[t], shared.at[dstv.at[0]], ssems[t]).wait()

  plsc.subcore_barrier()
  for part in range(NPARTS2):
    pltpu.sync_copy(shared.at[pl.ds(s * RPT + part * KE, KE)], b0)
    pltpu.sync_copy(b0, out_hbm.at[pl.ds(c * NP + s * RPT + part * KE, KE)])


_spmm128 = functools.partial(
    pl.kernel,
    out_type=jax.ShapeDtypeStruct((NC * NP, D_HID), jnp.float32),
    mesh=_mesh,
    scratch_types=[
        pltpu.VMEM((CHT // 4, 2 * KE), jnp.int32),
        pltpu.VMEM((PCH, KE), jnp.int32),
        pltpu.VMEM((KE, D_HID), jnp.float32),
        pltpu.VMEM((KE, D_HID), jnp.float32),
        pltpu.VMEM((KE, D_HID), jnp.float32),
        pltpu.VMEM((KE, D_HID), jnp.float32),
        pltpu.SemaphoreType.DMA,
        pltpu.SemaphoreType.DMA,
        pltpu.SemaphoreType.DMA,
        pltpu.SemaphoreType.DMA,
        pltpu.SemaphoreType.DMA,
        pltpu.SemaphoreType.DMA,
        pltpu.SemaphoreType.DMA,
        pltpu.SemaphoreType.DMA,
        pltpu.VMEM_SHARED((NP, D_HID), jnp.float32),
    ],
)(_spmm_body)


_BR = 512


def _dis_body(d0_ref, d1_ref, mask_ref, out_ref):
  deg = d0_ref[...] + d1_ref[...] + 1.0
  out_ref[...] = mask_ref[...] * lax.rsqrt(deg)


def _dis_kernel(d0, d1, mask):
  return pl.pallas_call(
      _dis_body,
      out_shape=jax.ShapeDtypeStruct((NP, 1), jnp.float32),
  )(d0, d1, mask)


def _pre_body(a_ref, w_ref, dis_ref, out_ref):
  h = jnp.dot(a_ref[...], w_ref[...], preferred_element_type=jnp.float32)
  out_ref[...] = h * dis_ref[...]


def _pre_kernel(a, w, dis):
  din, dout = w.shape
  return pl.pallas_call(
      _pre_body,
      grid=(NP // _BR,),
      in_specs=[
          pl.BlockSpec((_BR, din), lambda i: (i, 0)),
          pl.BlockSpec((din, dout), lambda i: (0, 0)),
          pl.BlockSpec((_BR, 1), lambda i: (i, 0)),
      ],
      out_specs=pl.BlockSpec((_BR, dout), lambda i: (i, 0)),
      out_shape=jax.ShapeDtypeStruct((NP, dout), jnp.float32),
  )(a, w, dis)


def _mid_body(z0_ref, z1_ref, xp_ref, dis_ref, b_ref, w_ref, out_ref):
  h = dis_ref[...] * (z0_ref[...] + z1_ref[...] + xp_ref[...]) + b_ref[...]
  a = jnp.maximum(h, 0.0)
  out_ref[...] = (
      jnp.dot(a, w_ref[...], preferred_element_type=jnp.float32)
      * dis_ref[...])


def _mid_kernel(z0, z1, xp, dis, b, w):
  din, dout = w.shape
  return pl.pallas_call(
      _mid_body,
      grid=(NP // _BR,),
      in_specs=[
          pl.BlockSpec((_BR, din), lambda i: (i, 0)),
          pl.BlockSpec((_BR, din), lambda i: (i, 0)),
          pl.BlockSpec((_BR, din), lambda i: (i, 0)),
          pl.BlockSpec((_BR, 1), lambda i: (i, 0)),
          pl.BlockSpec((1, din), lambda i: (0, 0)),
          pl.BlockSpec((din, dout), lambda i: (0, 0)),
      ],
      out_specs=pl.BlockSpec((_BR, dout), lambda i: (i, 0)),
      out_shape=jax.ShapeDtypeStruct((NP, dout), jnp.float32),
  )(z0, z1, xp, dis, b, w)


def _final_body(z0_ref, z1_ref, xp_ref, dis_ref, b_ref, out_ref):
  h = dis_ref[...] * (z0_ref[...] + z1_ref[...] + xp_ref[...]) + b_ref[...]
  nrm = jnp.sqrt(jnp.sum(h * h, axis=1, keepdims=True))
  out_ref[...] = h / jnp.maximum(nrm, 1e-12)


def _final_kernel(z0, z1, xp, dis, b):
  d = b.shape[1]
  return pl.pallas_call(
      _final_body,
      grid=(NP // _BR,),
      in_specs=[
          pl.BlockSpec((_BR, d), lambda i: (i, 0)),
          pl.BlockSpec((_BR, d), lambda i: (i, 0)),
          pl.BlockSpec((_BR, d), lambda i: (i, 0)),
          pl.BlockSpec((_BR, 1), lambda i: (i, 0)),
          pl.BlockSpec((1, d), lambda i: (0, 0)),
      ],
      out_specs=pl.BlockSpec((_BR, d), lambda i: (i, 0)),
      out_shape=jax.ShapeDtypeStruct((NP, d), jnp.float32),
  )(z0, z1, xp, dis, b)


def kernel(x, edge_index, W1, b1, W2, b2, W3, b3):
  ei = edge_index.astype(jnp.int32)
  pad = jnp.full((EP - N_EDGES,), N_NODES, jnp.int32)
  dstp_deg = jnp.concatenate([ei[1], pad]).reshape(NW, CH, K)
  srcp = jnp.concatenate([ei[0], pad]).reshape(NW, CHT // 2, 2 * KE)
  dstp = jnp.concatenate([ei[1], pad]).reshape(NW, CHT, KE)

  xpad = jnp.pad(x, ((0, NP - N_NODES), (0, 0)))
  mask = (jnp.arange(NP) < N_NODES).astype(jnp.float32).reshape(NP, 1)

  degp = _deg_kernel(dstp_deg)
  dis = _dis_kernel(degp[:NP, :1], degp[NP:, :1], mask)

  xp1 = _pre_kernel(xpad, W1, dis)
  zz = _spmm128(xp1, srcp, dstp)
  xp2 = _mid_kernel(zz[:NP], zz[NP:], xp1, dis, b1.reshape(1, -1), W2)
  zz = _spmm128(xp2, srcp, dstp)
  W3p = jnp.pad(W3, ((0, 0), (0, D_HID - D_EMB)))
  xp3 = _mid_kernel(zz[:NP], zz[NP:], xp2, dis, b2.reshape(1, -1), W3p)
  zz = _spmm128(xp3, srcp, dstp)
  emb = _final_kernel(zz[:NP, :D_EMB], zz[NP:, :D_EMB], xp3[:, :D_EMB],
                      dis, b3.reshape(1, -1))
  return emb[:N_NODES]
